# baseline (speedup 1.0000x reference)
import jax
import jax.numpy as jnp
from jax import lax
from jax.experimental import pallas as pl
from jax.experimental.pallas import tpu as pltpu
from jax.experimental.pallas import tpu_sc as plsc

NC, NS, L = 2, 16, 16
NW = NC * NS
B = 16384 * 200
D = 128
NROWS = 10
BPW = B // NW
UNIT = 128
U = BPW // UNIT
NBUF = 4
GRP = UNIT // L
FUNROLL = 4


def _body(idx_hbm, table_hbm, out_hbm, idx_v, table_v, rows_v, *sems):
    so = sems[0:NBUF]
    si = sems[NBUF:2 * NBUF]
    wid = lax.axis_index("s") * NC + lax.axis_index("c")
    idx_base = wid * U
    out_base = wid * BPW

    def idx_load(u, s):
        pltpu.async_copy(
            idx_hbm.at[pl.ds(idx_base + u, 1)], idx_v.at[pl.ds(s, 1)], si[s])

    def wait_idx(s):
        pltpu.make_async_copy(
            idx_hbm.at[pl.ds(0, 1)], idx_v.at[pl.ds(s, 1)], si[s]).wait()

    def out_copy(u, s):
        pltpu.async_copy(
            rows_v.at[pl.ds(s * UNIT, UNIT)],
            out_hbm.at[pl.ds(out_base + u * UNIT, UNIT)], so[s])

    def wait_out(s):
        pltpu.make_async_copy(
            rows_v.at[pl.ds(s * UNIT, UNIT)],
            out_hbm.at[pl.ds(0, UNIT)], so[s]).wait()

    pltpu.sync_copy(table_hbm, table_v)
    iota16 = lax.iota(jnp.int32, L)

    def compose(b):
        ridx = []
        tok = []
        for g in range(GRP):
            iv = idx_v[b, pl.ds(g * L, L)]
            ridx.append(jnp.minimum(jnp.maximum(iv, 0), NROWS - 1))
            tok.append(iota16 + (b * UNIT + g * L))

        def fbody(f, _):
            col = jnp.full((L,), 0, jnp.int32) + f
            for g in range(GRP):
                v = plsc.load_gather(table_v, [ridx[g], col])
                plsc.store_scatter(rows_v, [tok[g], col], v)
            return ()

        lax.fori_loop(0, D, fbody, (), unroll=FUNROLL)

    for s in range(NBUF):
        idx_load(s, s)

    def iter_body(i, _):
        for b in range(NBUF):
            u = i * NBUF + b
            wait_idx(s=b)
            @pl.when(i > 0)
            def _():
                wait_out(s=b)
            compose(b)
            out_copy(u, b)
            @pl.when(u + NBUF < U)
            def _():
                idx_load(u + NBUF, b)
        return ()

    lax.fori_loop(0, U // NBUF, iter_body, ())

    for s in range(NBUF):
        wait_out(s)


@jax.jit
def _emb(idx2d, table_flat):
    mesh = plsc.VectorSubcoreMesh(core_axis_name="c", subcore_axis_name="s")
    return pl.kernel(
        _body,
        out_type=jax.ShapeDtypeStruct((B, D), jnp.float32),
        mesh=mesh,
        compiler_params=pltpu.CompilerParams(needs_layout_passes=False),
        scratch_types=[
            pltpu.VMEM((NBUF, UNIT), jnp.int32),
            pltpu.VMEM((NROWS, D), jnp.float32),
            pltpu.VMEM((NBUF * UNIT, D), jnp.float32),
        ] + [pltpu.SemaphoreType.DMA] * (2 * NBUF),
    )(idx2d, table_flat)


def kernel(tokentypes, emb_weight):
    idx2d = jnp.reshape(tokentypes.astype(jnp.int32), (B // UNIT, UNIT))
    out = _emb(idx2d, emb_weight)
    return jnp.reshape(out, (16384, 200, D))


# speedup vs baseline: 2.4238x; 2.4238x over previous
import jax
import jax.numpy as jnp
from jax import lax
from jax.experimental import pallas as pl
from jax.experimental.pallas import tpu as pltpu
from jax.experimental.pallas import tpu_sc as plsc

NC, NS, L = 2, 16, 16
NW = NC * NS
B = 16384 * 200
D = 128
NROWS = 10
BPW = B // NW
UNIT = 128
U = BPW // UNIT
NBUF = 4
GRP = UNIT // L
FUNROLL = 4


def _body(idx_hbm, table_hbm, out_hbm, idx_v, table_v, rows_v, *sems):
    so = sems[0:NBUF]
    si = sems[NBUF:2 * NBUF]
    wid = lax.axis_index("s") * NC + lax.axis_index("c")
    idx_base = wid * U
    out_base = wid * BPW

    def idx_load(u, s):
        pltpu.async_copy(
            idx_hbm.at[pl.ds(idx_base + u, 1)], idx_v.at[pl.ds(s, 1)], si[s])

    def wait_idx(s):
        pltpu.make_async_copy(
            idx_hbm.at[pl.ds(0, 1)], idx_v.at[pl.ds(s, 1)], si[s]).wait()

    def out_copy(u, s):
        pltpu.async_copy(
            rows_v.at[pl.ds(s * UNIT, UNIT)],
            out_hbm.at[pl.ds(out_base + u * UNIT, UNIT)], so[s])

    def wait_out(s):
        pltpu.make_async_copy(
            rows_v.at[pl.ds(s * UNIT, UNIT)],
            out_hbm.at[pl.ds(0, UNIT)], so[s]).wait()

    pltpu.sync_copy(table_hbm, table_v)
    iota16 = lax.iota(jnp.int32, L)

    def compose(b):
        ridx = []
        tok = []
        for g in range(GRP):
            iv = idx_v[b, pl.ds(g * L, L)]
            ridx.append(jnp.minimum(jnp.maximum(iv, 0), NROWS - 1))
            tok.append(iota16 + (b * UNIT + g * L))

        @plsc.parallel_loop(0, D, unroll=FUNROLL)
        def fbody(f):
            col = jnp.full((L,), 0, jnp.int32) + f
            for g in range(GRP):
                v = plsc.load_gather(table_v, [ridx[g], col])
                plsc.store_scatter(rows_v, [tok[g], col], v)

    for s in range(NBUF):
        idx_load(s, s)

    def iter_body(i, _):
        for b in range(NBUF):
            u = i * NBUF + b
            wait_idx(s=b)
            @pl.when(i > 0)
            def _():
                wait_out(s=b)
            compose(b)
            out_copy(u, b)
            @pl.when(u + NBUF < U)
            def _():
                idx_load(u + NBUF, b)
        return ()

    lax.fori_loop(0, U // NBUF, iter_body, ())

    for s in range(NBUF):
        wait_out(s)


@jax.jit
def _emb(idx2d, table_flat):
    mesh = plsc.VectorSubcoreMesh(core_axis_name="c", subcore_axis_name="s")
    return pl.kernel(
        _body,
        out_type=jax.ShapeDtypeStruct((B, D), jnp.float32),
        mesh=mesh,
        compiler_params=pltpu.CompilerParams(needs_layout_passes=False),
        scratch_types=[
            pltpu.VMEM((NBUF, UNIT), jnp.int32),
            pltpu.VMEM((NROWS, D), jnp.float32),
            pltpu.VMEM((NBUF * UNIT, D), jnp.float32),
        ] + [pltpu.SemaphoreType.DMA] * (2 * NBUF),
    )(idx2d, table_flat)


def kernel(tokentypes, emb_weight):
    idx2d = jnp.reshape(tokentypes.astype(jnp.int32), (B // UNIT, UNIT))
    out = _emb(idx2d, emb_weight)
    return jnp.reshape(out, (16384, 200, D))


# speedup vs baseline: 24.5130x; 10.1136x over previous
import jax
import jax.numpy as jnp
from jax import lax
from jax.experimental import pallas as pl
from jax.experimental.pallas import tpu as pltpu
from jax.experimental.pallas import tpu_sc as plsc

NC, NS, L = 2, 16, 16
NW = NC * NS
B = 16384 * 200
D = 128
NROWS = 10
BPW = B // NW
UNIT = 128
U = BPW // UNIT
NBUF = 4
GRP = UNIT // L
FUNROLL = 4


def _body(idx_hbm, table_hbm, out_hbm, idx_v, table_v, rows_v, *sems):
    so = sems[0:NBUF]
    si = sems[NBUF:2 * NBUF]
    wid = lax.axis_index("s") * NC + lax.axis_index("c")
    idx_base = wid * U
    out_base = wid * BPW

    def idx_load(u, s):
        pltpu.async_copy(
            idx_hbm.at[pl.ds(idx_base + u, 1)],
            idx_v.at[pl.ds(s, 1), pl.ds(0, UNIT)], si[s])

    def wait_idx(s):
        pltpu.make_async_copy(
            idx_hbm.at[pl.ds(0, 1)],
            idx_v.at[pl.ds(s, 1), pl.ds(0, UNIT)], si[s]).wait()

    def out_copy(u, s):
        pltpu.async_copy(
            rows_v.at[pl.ds(s * UNIT, UNIT)],
            out_hbm.at[pl.ds(out_base + u * UNIT, UNIT)], so[s])

    def wait_out(s):
        pltpu.make_async_copy(
            rows_v.at[pl.ds(s * UNIT, UNIT)],
            out_hbm.at[pl.ds(0, UNIT)], so[s]).wait()

    pltpu.sync_copy(table_hbm, table_v)
    iota16 = lax.iota(jnp.int32, L)

    def compose(b):
        @plsc.parallel_loop(0, UNIT, unroll=FUNROLL)
        def tbody(t):
            sidx = idx_v[b, pl.ds(t, L)][0]
            sidx = jnp.minimum(jnp.maximum(sidx, 0), NROWS - 1)
            for k in range(D // L):
                v = table_v[sidx, pl.ds(k * L, L)]
                rows_v[b * UNIT + t, pl.ds(k * L, L)] = v

    for s in range(NBUF):
        idx_load(s, s)

    def iter_body(i, _):
        for b in range(NBUF):
            u = i * NBUF + b
            wait_idx(s=b)
            @pl.when(i > 0)
            def _():
                wait_out(s=b)
            compose(b)
            out_copy(u, b)
            @pl.when(u + NBUF < U)
            def _():
                idx_load(u + NBUF, b)
        return ()

    lax.fori_loop(0, U // NBUF, iter_body, ())

    for s in range(NBUF):
        wait_out(s)


@jax.jit
def _emb(idx2d, table_flat):
    mesh = plsc.VectorSubcoreMesh(core_axis_name="c", subcore_axis_name="s")
    return pl.kernel(
        _body,
        out_type=jax.ShapeDtypeStruct((B, D), jnp.float32),
        mesh=mesh,
        compiler_params=pltpu.CompilerParams(needs_layout_passes=False),
        scratch_types=[
            pltpu.VMEM((NBUF, UNIT + L), jnp.int32),
            pltpu.VMEM((NROWS, D), jnp.float32),
            pltpu.VMEM((NBUF * UNIT, D), jnp.float32),
        ] + [pltpu.SemaphoreType.DMA] * (2 * NBUF),
    )(idx2d, table_flat)


def kernel(tokentypes, emb_weight):
    idx2d = jnp.reshape(tokentypes.astype(jnp.int32), (B // UNIT, UNIT))
    out = _emb(idx2d, emb_weight)
    return jnp.reshape(out, (16384, 200, D))
